# Initial kernel scaffold; baseline (speedup 1.0000x reference)
import jax
import jax.numpy as jnp
from jax.experimental import pallas as pl


def kernel(cls_heads, reg_heads, batch_anchors):
    raise NotImplementedError("write your pallas kernel here")


# speedup vs baseline: 1535.2192x; 1535.2192x over previous
import jax
import jax.numpy as jnp
from jax import lax
from jax.experimental import pallas as pl

_IMAGE_W = 1024
_IMAGE_H = 1024
_MIN_SCORE = 0.05
_NMS_THR = 0.5
_MAX_DET = 100
_LANES = 128
_NEG_INF = float("-inf")


def _decoder_body(cls_ref, reg_ref, anc_ref, s_out_ref, c_out_ref, b_out_ref):
    B, C, R, L = cls_ref.shape
    N = R * L

    def class_step(c, carry):
        m, idx = carry
        v = cls_ref[:, c]
        better = v > m
        return jnp.where(better, v, m), jnp.where(better, c, idx)

    m0 = cls_ref[:, 0]
    idx0 = jnp.zeros((B, R, L), jnp.int32)
    scores, classes = lax.fori_loop(1, C, class_step, (m0, idx0))
    classes_f = classes.astype(jnp.float32)

    reg = reg_ref[...]
    anc = anc_ref[...]
    ax1, ay1, ax2, ay2 = anc[:, 0], anc[:, 1], anc[:, 2], anc[:, 3]
    aw = ax2 - ax1
    ah = ay2 - ay1
    acx = ax1 + 0.5 * aw
    acy = ay1 + 0.5 * ah
    tx = reg[:, 0] * 0.1
    ty = reg[:, 1] * 0.1
    tw = reg[:, 2] * 0.2
    th = reg[:, 3] * 0.2
    w = jnp.exp(tw) * aw
    h = jnp.exp(th) * ah
    cx = tx * aw + acx
    cy = ty * ah + acy
    bx1 = jnp.maximum((cx - 0.5 * w).astype(jnp.int32), 0).astype(jnp.float32)
    by1 = jnp.maximum((cy - 0.5 * h).astype(jnp.int32), 0).astype(jnp.float32)
    bx2 = jnp.minimum((cx + 0.5 * w).astype(jnp.int32), _IMAGE_W - 1).astype(jnp.float32)
    by2 = jnp.minimum((cy + 0.5 * h).astype(jnp.int32), _IMAGE_H - 1).astype(jnp.float32)
    areas = (bx2 - bx1) * (by2 - by1)

    work = jnp.where(scores > _MIN_SCORE, scores, _NEG_INF)
    li = (lax.broadcasted_iota(jnp.int32, (B, R, L), 1) * L
          + lax.broadcasted_iota(jnp.int32, (B, R, L), 2))
    lane = lax.broadcasted_iota(jnp.int32, (B, _LANES), 1)

    def step(i, carry):
        work, so, co, o0, o1, o2, o3 = carry
        m = jnp.max(work, axis=(1, 2), keepdims=True)
        pick = jnp.min(jnp.where(work == m, li, N), axis=(1, 2), keepdims=True)
        oh = li == pick

        def sel(x):
            return jnp.sum(jnp.where(oh, x, 0.0), axis=(1, 2), keepdims=True)

        cm = sel(classes_f)
        px1 = sel(bx1)
        py1 = sel(by1)
        px2 = sel(bx2)
        py2 = sel(by2)
        pa = sel(areas)
        xx1 = jnp.maximum(px1, bx1)
        yy1 = jnp.maximum(py1, by1)
        xx2 = jnp.minimum(px2, bx2)
        yy2 = jnp.minimum(py2, by2)
        iw = jnp.maximum(xx2 - xx1, 0.0)
        ih = jnp.maximum(yy2 - yy1, 0.0)
        inter = iw * ih
        union = pa + areas - inter
        iou = jnp.where(union > 0, inter / jnp.where(union > 0, union, 1.0), 0.0)
        sup = (iou >= _NMS_THR) & (classes_f == cm)
        valid = m > _NEG_INF
        work = jnp.where(valid & (sup | oh), _NEG_INF, work)

        pos = (lane == i) & valid[:, :, 0]
        so = jnp.where(pos, m[:, :, 0], so)
        co = jnp.where(pos, cm[:, :, 0], co)
        o0 = jnp.where(pos, px1[:, :, 0], o0)
        o1 = jnp.where(pos, py1[:, :, 0], o1)
        o2 = jnp.where(pos, px2[:, :, 0], o2)
        o3 = jnp.where(pos, py2[:, :, 0], o3)
        return work, so, co, o0, o1, o2, o3

    neg1 = jnp.full((B, _LANES), -1.0, jnp.float32)
    carry = lax.fori_loop(0, _MAX_DET, step,
                          (work, neg1, neg1, neg1, neg1, neg1, neg1))
    _, so, co, o0, o1, o2, o3 = carry
    s_out_ref[...] = so
    c_out_ref[...] = co
    b_out_ref[:, 0, :] = o0
    b_out_ref[:, 1, :] = o1
    b_out_ref[:, 2, :] = o2
    b_out_ref[:, 3, :] = o3


def _run_decoder(cls4, reg4, anc4):
    B = cls4.shape[0]
    return pl.pallas_call(
        _decoder_body,
        out_shape=[
            jax.ShapeDtypeStruct((B, _LANES), jnp.float32),
            jax.ShapeDtypeStruct((B, _LANES), jnp.float32),
            jax.ShapeDtypeStruct((B, 4, _LANES), jnp.float32),
        ],
    )(cls4, reg4, anc4)


def kernel(cls_heads, reg_heads, batch_anchors):
    cls = jnp.concatenate([cls_heads[i] for i in range(cls_heads.shape[0])], axis=1)
    reg = jnp.concatenate([reg_heads[i] for i in range(reg_heads.shape[0])], axis=1)
    anc = jnp.concatenate([batch_anchors[i] for i in range(batch_anchors.shape[0])], axis=1)
    B, N, C = cls.shape
    NP = -(-N // _LANES) * _LANES
    R = NP // _LANES
    clsT = jnp.pad(jnp.transpose(cls, (0, 2, 1)),
                   ((0, 0), (0, 0), (0, NP - N)), constant_values=-1.0)
    regT = jnp.pad(jnp.transpose(reg, (0, 2, 1)), ((0, 0), (0, 0), (0, NP - N)))
    ancT = jnp.pad(jnp.transpose(anc, (0, 2, 1)), ((0, 0), (0, 0), (0, NP - N)))
    so, co, bo = _run_decoder(clsT.reshape(B, C, R, _LANES),
                              regT.reshape(B, 4, R, _LANES),
                              ancT.reshape(B, 4, R, _LANES))
    s = so[:, :_MAX_DET]
    c = co[:, :_MAX_DET]
    b = jnp.transpose(bo, (0, 2, 1))[:, :_MAX_DET, :]
    return s, c, b
